# Initial kernel scaffold; baseline (speedup 1.0000x reference)
import jax
import jax.numpy as jnp
from jax.experimental import pallas as pl


def kernel(h, x, n_atoms, params):
    raise NotImplementedError("write your pallas kernel here")


# speedup vs baseline: 160.7970x; 160.7970x over previous
import jax
import jax.numpy as jnp
from jax import lax
from jax.experimental import pallas as pl

_N = 48
_E = _N * _N
_DH = 128
_DV = 64
_H = 8
_dh = _DH // _H
_dvh = _DV // _H
_NRBF = 16
_L = 4


def _se3_body(h_ref, x_ref, win_ref, wq_ref, wk_ref, wv_ref, wo_ref,
              bo_ref, wr1_ref, br1_ref, wrk_ref, wrv_ref, g_ref, b_ref,
              wfin_ref, bfin_ref, out_ref):
    f32 = jnp.float32
    h = h_ref[0]
    feats = jnp.dot(h, win_ref[...], preferred_element_type=f32)

    x = x_ref[0]
    xi = jnp.broadcast_to(x[:, None, :], (_N, _N, 3))
    xj = jnp.broadcast_to(x[None, :, :], (_N, _N, 3))
    diff = xj - xi
    d2 = jnp.sum(diff * diff, axis=2, keepdims=True)
    dist = jnp.sqrt(d2.reshape(_E, 1) + 1e-12)
    centers = lax.broadcasted_iota(jnp.int32, (1, _NRBF), 1).astype(f32) * (
        5.0 / (_NRBF - 1))
    rb = jnp.exp(-4.0 * (dist - centers) ** 2)

    di = lax.broadcasted_iota(jnp.int32, (_DH, _H), 0)
    hi = lax.broadcasted_iota(jnp.int32, (_DH, _H), 1)
    s_sum = (di // _dh == hi).astype(f32)
    ci = lax.broadcasted_iota(jnp.int32, (_H, _DV), 1)
    hi2 = lax.broadcasted_iota(jnp.int32, (_H, _DV), 0)
    s_exp = (ci // _dvh == hi2).astype(f32)

    ii = lax.broadcasted_iota(jnp.int32, (_N, _N, _H), 0)
    jj = lax.broadcasted_iota(jnp.int32, (_N, _N, _H), 1)
    diag = ii == jj

    inv_sqrt_dh = 1.0 / (_dh ** 0.5)

    for l in range(_L):
        q = jnp.dot(feats, wq_ref[l], preferred_element_type=f32)
        k = jnp.dot(feats, wk_ref[l], preferred_element_type=f32)
        v = jnp.dot(feats, wv_ref[l], preferred_element_type=f32)

        rh = jnp.maximum(
            jnp.dot(rb, wr1_ref[l], preferred_element_type=f32) + br1_ref[l],
            0.0)
        rk = jnp.dot(rh, wrk_ref[l], preferred_element_type=f32)
        rv = jnp.dot(rh, wrv_ref[l], preferred_element_type=f32)

        qe = jnp.broadcast_to(q[:, None, :], (_N, _N, _DH)).reshape(_E, _DH)
        ke = jnp.broadcast_to(k[None, :, :], (_N, _N, _DH)).reshape(_E, _DH)
        prod = qe * ke
        logits = jnp.dot(prod, s_sum, preferred_element_type=f32)
        logits = logits * rk * inv_sqrt_dh

        l3 = logits.reshape(_N, _N, _H)
        l3 = jnp.where(diag, -1e30, l3)
        mx = jnp.max(l3, axis=1, keepdims=True)
        ex = jnp.exp(l3 - mx)
        den = jnp.sum(ex, axis=1, keepdims=True)
        alpha = ex / (den + 1e-9)

        w3 = (alpha * rv.reshape(_N, _N, _H)).reshape(_E, _H)
        we = jnp.dot(w3, s_exp, preferred_element_type=f32)
        ve = jnp.broadcast_to(v[None, :, :], (_N, _N, _DV)).reshape(_E, _DV)
        agg = jnp.sum((we * ve).reshape(_N, _N, _DV), axis=1)

        feats = feats + jnp.dot(agg, wo_ref[l],
                                preferred_element_type=f32) + bo_ref[l]
        mu = jnp.mean(feats, axis=-1, keepdims=True)
        xc = feats - mu
        var = jnp.mean(xc * xc, axis=-1, keepdims=True)
        feats = xc / jnp.sqrt(var + 1e-5) * g_ref[l] + b_ref[l]

    out = jnp.dot(feats, wfin_ref[...], preferred_element_type=f32)
    out = out + bfin_ref[...]
    out_ref[0] = jnp.sum(out, axis=0, keepdims=True)


def kernel(h, x, n_atoms, params):
    Bsz = h.shape[0]
    f32 = jnp.float32
    layers = params["layers"]

    def stk(name):
        return jnp.stack([p[name] for p in layers])

    wq, wk, wv, wo = stk("Wq"), stk("Wk"), stk("Wv"), stk("Wo")
    bo = stk("bo").reshape(_L, 1, _DH)
    wr1 = stk("Wr1")
    br1 = stk("br1").reshape(_L, 1, -1)
    wrk, wrv = stk("Wrk"), stk("Wrv")
    gamma = stk("gamma").reshape(_L, 1, _DH)
    beta = stk("beta").reshape(_L, 1, _DH)
    bfin = params["b_fin"].reshape(1, _DH)

    def full(shape):
        return pl.BlockSpec(shape, lambda b: (0,) * len(shape))

    grid_spec = pl.GridSpec(
        grid=(Bsz,),
        in_specs=[
            pl.BlockSpec((1, _N, h.shape[2]), lambda b: (b, 0, 0)),
            pl.BlockSpec((1, _N, 3), lambda b: (b, 0, 0)),
            full(params["W_in"].shape),
            full(wq.shape), full(wk.shape), full(wv.shape), full(wo.shape),
            full(bo.shape), full(wr1.shape), full(br1.shape),
            full(wrk.shape), full(wrv.shape), full(gamma.shape),
            full(beta.shape), full(params["W_fin"].shape), full(bfin.shape),
        ],
        out_specs=pl.BlockSpec((1, 1, _DH), lambda b: (b, 0, 0)),
    )

    out = pl.pallas_call(
        _se3_body,
        grid_spec=grid_spec,
        out_shape=jax.ShapeDtypeStruct((Bsz, 1, _DH), f32),
    )(h, x, params["W_in"], wq, wk, wv, wo, bo, wr1, br1, wrk, wrv,
      gamma, beta, params["W_fin"], bfin)

    return out.reshape(Bsz, _DH) / jnp.asarray(n_atoms, f32)


# speedup vs baseline: 206.7141x; 1.2856x over previous
import jax
import jax.numpy as jnp
from jax import lax
from jax.experimental import pallas as pl

_N = 48
_E = _N * _N
_DH = 128
_DV = 64
_H = 8
_dh = _DH // _H
_dvh = _DV // _H
_NRBF = 16
_L = 4
_G = 1


def _one_molecule(h, x, win_ref, wq_ref, wk_ref, wv_ref, wo_ref, bo_ref,
                  w1cat_ref, b1cat_ref, wbd_ref, g_ref, b_ref,
                  wfin_ref, bfin_ref, diag, centers):
    f32 = jnp.float32
    feats = jnp.dot(h, win_ref[...], preferred_element_type=f32)

    xi = jnp.broadcast_to(x[:, None, :], (_N, _N, 3))
    xj = jnp.broadcast_to(x[None, :, :], (_N, _N, 3))
    diff = xj - xi
    d2 = jnp.sum(diff * diff, axis=2, keepdims=True)
    dist = jnp.sqrt(d2.reshape(_E, 1) + 1e-12)
    rb = jnp.exp(-4.0 * (dist - centers) ** 2)

    rhid = jnp.maximum(
        jnp.dot(rb, w1cat_ref[...], preferred_element_type=f32)
        + b1cat_ref[...], 0.0)
    rkv = jnp.dot(rhid, wbd_ref[...], preferred_element_type=f32)
    rkv_st = jnp.transpose(rkv.reshape(_N, _N, _L * 2 * _H),
                           (2, 0, 1)).reshape(_L * 2 * _H * _N, _N)

    inv_sqrt_dh = 1.0 / (_dh ** 0.5)

    for l in range(_L):
        q = jnp.dot(feats, wq_ref[l], preferred_element_type=f32)
        k = jnp.dot(feats, wk_ref[l], preferred_element_type=f32)
        v = jnp.dot(feats, wv_ref[l], preferred_element_type=f32)
        kt = jnp.transpose(k)

        lst = jnp.concatenate(
            [jnp.dot(q[:, hd * _dh:(hd + 1) * _dh],
                     kt[hd * _dh:(hd + 1) * _dh, :],
                     preferred_element_type=f32) for hd in range(_H)],
            axis=0)
        base = l * 2 * _H * _N
        rkst = rkv_st[base:base + _H * _N]
        rvst = rkv_st[base + _H * _N:base + 2 * _H * _N]
        lst = lst * inv_sqrt_dh * rkst
        lst = jnp.where(diag, -1e30, lst)
        mx = jnp.max(lst, axis=1, keepdims=True)
        ex = jnp.exp(lst - mx)
        den = jnp.sum(ex, axis=1, keepdims=True)
        ast = ex / (den + 1e-9) * rvst
        agg = jnp.concatenate(
            [jnp.dot(ast[hd * _N:(hd + 1) * _N],
                     v[:, hd * _dvh:(hd + 1) * _dvh],
                     preferred_element_type=f32) for hd in range(_H)],
            axis=1)

        feats = feats + jnp.dot(agg, wo_ref[l],
                                preferred_element_type=f32) + bo_ref[l]
        mu = jnp.mean(feats, axis=-1, keepdims=True)
        xc = feats - mu
        var = jnp.mean(xc * xc, axis=-1, keepdims=True)
        feats = xc / jnp.sqrt(var + 1e-5) * g_ref[l] + b_ref[l]

    out = jnp.dot(feats, wfin_ref[...], preferred_element_type=f32)
    out = out + bfin_ref[...]
    return jnp.sum(out, axis=0, keepdims=True)


def _se3_body(h_ref, x_ref, win_ref, wq_ref, wk_ref, wv_ref, wo_ref,
              bo_ref, w1cat_ref, b1cat_ref, wbd_ref, g_ref, b_ref,
              wfin_ref, bfin_ref, out_ref):
    f32 = jnp.float32
    centers = lax.broadcasted_iota(jnp.int32, (1, _NRBF), 1).astype(f32) * (
        5.0 / (_NRBF - 1))
    si = lax.broadcasted_iota(jnp.int32, (_H * _N, _N), 0)
    sj = lax.broadcasted_iota(jnp.int32, (_H * _N, _N), 1)
    diag = (si % _N) == sj
    for g in range(_G):
        out_ref[g] = _one_molecule(
            h_ref[g], x_ref[g], win_ref, wq_ref, wk_ref, wv_ref, wo_ref,
            bo_ref, w1cat_ref, b1cat_ref, wbd_ref, g_ref, b_ref,
            wfin_ref, bfin_ref, diag, centers)


def kernel(h, x, n_atoms, params):
    Bsz = h.shape[0]
    f32 = jnp.float32
    layers = params["layers"]

    def stk(name):
        return jnp.stack([p[name] for p in layers])

    wq, wk, wv, wo = stk("Wq"), stk("Wk"), stk("Wv"), stk("Wo")
    bo = stk("bo").reshape(_L, 1, _DH)
    w1cat = jnp.concatenate([p["Wr1"] for p in layers], axis=1)
    b1cat = jnp.concatenate([p["br1"] for p in layers]).reshape(1, -1)
    rhid_n = layers[0]["Wr1"].shape[1]
    wbd = jnp.zeros((_L * rhid_n, _L * 2 * _H), f32)
    for l, p in enumerate(layers):
        blk = jnp.concatenate([p["Wrk"], p["Wrv"]], axis=1)
        wbd = wbd.at[l * rhid_n:(l + 1) * rhid_n,
                     l * 2 * _H:(l + 1) * 2 * _H].set(blk)
    gamma = stk("gamma").reshape(_L, 1, _DH)
    beta = stk("beta").reshape(_L, 1, _DH)
    bfin = params["b_fin"].reshape(1, _DH)

    def full(shape):
        return pl.BlockSpec(shape, lambda b: (0,) * len(shape))

    grid_spec = pl.GridSpec(
        grid=(Bsz // _G,),
        in_specs=[
            pl.BlockSpec((_G, _N, h.shape[2]), lambda b: (b, 0, 0)),
            pl.BlockSpec((_G, _N, 3), lambda b: (b, 0, 0)),
            full(params["W_in"].shape),
            full(wq.shape), full(wk.shape), full(wv.shape), full(wo.shape),
            full(bo.shape), full(w1cat.shape), full(b1cat.shape),
            full(wbd.shape), full(gamma.shape),
            full(beta.shape), full(params["W_fin"].shape), full(bfin.shape),
        ],
        out_specs=pl.BlockSpec((_G, 1, _DH), lambda b: (b, 0, 0)),
    )

    out = pl.pallas_call(
        _se3_body,
        grid_spec=grid_spec,
        out_shape=jax.ShapeDtypeStruct((Bsz, 1, _DH), f32),
    )(h, x, params["W_in"], wq, wk, wv, wo, bo, w1cat, b1cat, wbd,
      gamma, beta, params["W_fin"], bfin)

    return out.reshape(Bsz, _DH) / jnp.asarray(n_atoms, f32)


# speedup vs baseline: 320.4022x; 1.5500x over previous
import jax
import jax.numpy as jnp
from jax import lax
from jax.experimental import pallas as pl

_N = 48
_E = _N * _N
_DH = 128
_DV = 64
_H = 8
_dh = _DH // _H
_dvh = _DV // _H
_NRBF = 16
_L = 4
_G = 4


def _se3_body(h_ref, x_ref, win_ref, wq_ref, wk_ref, wv_ref, wo_ref,
              bo_ref, w1cat_ref, b1cat_ref, wbd_ref, g_ref, b_ref,
              wfin_ref, bfin_ref, out_ref):
    f32 = jnp.float32
    ng = _G * _N
    sh = _H * _N
    centers = lax.broadcasted_iota(jnp.int32, (1, _NRBF), 1).astype(f32) * (
        5.0 / (_NRBF - 1))
    si = lax.broadcasted_iota(jnp.int32, (_G * sh, _N), 0)
    sj = lax.broadcasted_iota(jnp.int32, (_G * sh, _N), 1)
    diag = (si % _N) == sj
    hm0 = lax.broadcasted_iota(jnp.int32, (sh, _DH), 0)
    hm1 = lax.broadcasted_iota(jnp.int32, (sh, _DH), 1)
    headmask = (hm0 // _N == hm1 // _dh).astype(f32)
    sg0 = lax.broadcasted_iota(jnp.int32, (_G, ng), 0)
    sg1 = lax.broadcasted_iota(jnp.int32, (_G, ng), 1)
    sumsel = (sg1 // _N == sg0).astype(f32)

    feats = jnp.dot(h_ref[...].reshape(ng, -1), win_ref[...],
                    preferred_element_type=f32)

    x4 = x_ref[...]
    xi = jnp.broadcast_to(x4[:, :, None, :], (_G, _N, _N, 3))
    xj = jnp.broadcast_to(x4[:, None, :, :], (_G, _N, _N, 3))
    diff = xj - xi
    d2 = jnp.sum(diff * diff, axis=3, keepdims=True)
    dist = jnp.sqrt(d2.reshape(_G * _E, 1) + 1e-12)
    rb = jnp.exp(-4.0 * (dist - centers) ** 2)

    rhid = jnp.maximum(
        jnp.dot(rb, w1cat_ref[...], preferred_element_type=f32)
        + b1cat_ref[...], 0.0)
    rkv = jnp.dot(rhid, wbd_ref[...], preferred_element_type=f32)
    rkv4 = rkv.reshape(_G, _N, _N, _L * 2 * _H)
    sts = [jnp.transpose(rkv4[g], (2, 0, 1)).reshape(_L * 2 * sh, _N)
           for g in range(_G)]

    inv_sqrt_dh = 1.0 / (_dh ** 0.5)

    for l in range(_L):
        q = jnp.dot(feats, wq_ref[l], preferred_element_type=f32)
        k = jnp.dot(feats, wk_ref[l], preferred_element_type=f32)
        v = jnp.dot(feats, wv_ref[l], preferred_element_type=f32)

        lst = jnp.concatenate(
            [jnp.dot(
                jnp.broadcast_to(q[g * _N:(g + 1) * _N][None],
                                 (_H, _N, _DH)).reshape(sh, _DH) * headmask,
                jnp.transpose(k[g * _N:(g + 1) * _N]),
                preferred_element_type=f32) for g in range(_G)],
            axis=0)
        base = l * 2 * sh
        rkst = jnp.concatenate(
            [sts[g][base:base + sh] for g in range(_G)], axis=0)
        rvst = jnp.concatenate(
            [sts[g][base + sh:base + 2 * sh] for g in range(_G)], axis=0)
        lst = lst * inv_sqrt_dh * rkst
        lst = jnp.where(diag, -1e30, lst)
        mx = jnp.max(lst, axis=1, keepdims=True)
        ex = jnp.exp(lst - mx)
        den = jnp.sum(ex, axis=1, keepdims=True)
        ast = ex / (den + 1e-9) * rvst

        agg = jnp.concatenate(
            [jnp.concatenate(
                [jnp.dot(ast[g * sh + hd * _N:g * sh + (hd + 1) * _N],
                         v[g * _N:(g + 1) * _N,
                           hd * _dvh:(hd + 1) * _dvh],
                         preferred_element_type=f32) for hd in range(_H)],
                axis=1) for g in range(_G)],
            axis=0)

        feats = feats + jnp.dot(agg, wo_ref[l],
                                preferred_element_type=f32) + bo_ref[l]
        mu = jnp.mean(feats, axis=-1, keepdims=True)
        xc = feats - mu
        var = jnp.mean(xc * xc, axis=-1, keepdims=True)
        feats = xc / jnp.sqrt(var + 1e-5) * g_ref[l] + b_ref[l]

    out = jnp.dot(feats, wfin_ref[...], preferred_element_type=f32)
    out = out + bfin_ref[...]
    res = jnp.dot(sumsel, out, preferred_element_type=f32)
    for g in range(_G):
        out_ref[g] = res[g:g + 1]


def kernel(h, x, n_atoms, params):
    Bsz = h.shape[0]
    f32 = jnp.float32
    layers = params["layers"]

    def stk(name):
        return jnp.stack([p[name] for p in layers])

    wq, wk, wv, wo = stk("Wq"), stk("Wk"), stk("Wv"), stk("Wo")
    bo = stk("bo").reshape(_L, 1, _DH)
    w1cat = jnp.concatenate([p["Wr1"] for p in layers], axis=1)
    b1cat = jnp.concatenate([p["br1"] for p in layers]).reshape(1, -1)
    rhid_n = layers[0]["Wr1"].shape[1]
    wbd = jnp.zeros((_L * rhid_n, _L * 2 * _H), f32)
    for l, p in enumerate(layers):
        blk = jnp.concatenate([p["Wrk"], p["Wrv"]], axis=1)
        wbd = wbd.at[l * rhid_n:(l + 1) * rhid_n,
                     l * 2 * _H:(l + 1) * 2 * _H].set(blk)
    gamma = stk("gamma").reshape(_L, 1, _DH)
    beta = stk("beta").reshape(_L, 1, _DH)
    bfin = params["b_fin"].reshape(1, _DH)

    def full(shape):
        return pl.BlockSpec(shape, lambda b: (0,) * len(shape))

    grid_spec = pl.GridSpec(
        grid=(Bsz // _G,),
        in_specs=[
            pl.BlockSpec((_G, _N, h.shape[2]), lambda b: (b, 0, 0)),
            pl.BlockSpec((_G, _N, 3), lambda b: (b, 0, 0)),
            full(params["W_in"].shape),
            full(wq.shape), full(wk.shape), full(wv.shape), full(wo.shape),
            full(bo.shape), full(w1cat.shape), full(b1cat.shape),
            full(wbd.shape), full(gamma.shape),
            full(beta.shape), full(params["W_fin"].shape), full(bfin.shape),
        ],
        out_specs=pl.BlockSpec((_G, 1, _DH), lambda b: (b, 0, 0)),
    )

    out = pl.pallas_call(
        _se3_body,
        grid_spec=grid_spec,
        out_shape=jax.ShapeDtypeStruct((Bsz, 1, _DH), f32),
    )(h, x, params["W_in"], wq, wk, wv, wo, bo, w1cat, b1cat, wbd,
      gamma, beta, params["W_fin"], bfin)

    return out.reshape(Bsz, _DH) / jnp.asarray(n_atoms, f32)


# speedup vs baseline: 372.1029x; 1.1614x over previous
import jax
import jax.numpy as jnp
from jax import lax
from jax.experimental import pallas as pl

_N = 48
_E = _N * _N
_DH = 128
_DV = 64
_H = 8
_dh = _DH // _H
_dvh = _DV // _H
_NRBF = 16
_L = 4
_G = 8


def _se3_body(h_ref, x_ref, win_ref, wq_ref, wk_ref, wv_ref, wo_ref,
              bo_ref, w1cat_ref, b1cat_ref, wbd_ref, g_ref, b_ref,
              wfin_ref, bfin_ref, out_ref):
    f32 = jnp.float32
    ng = _G * _N
    sh = _H * _N
    centers = lax.broadcasted_iota(jnp.int32, (1, _NRBF), 1).astype(f32) * (
        5.0 / (_NRBF - 1))
    si = lax.broadcasted_iota(jnp.int32, (_G * sh, _N), 0)
    sj = lax.broadcasted_iota(jnp.int32, (_G * sh, _N), 1)
    negdiag = jnp.where((si % _N) == sj, -1e30, 0.0)
    hm0 = lax.broadcasted_iota(jnp.int32, (sh, _DH), 0)
    hm1 = lax.broadcasted_iota(jnp.int32, (sh, _DH), 1)
    headmask = (hm0 // _N == hm1 // _dh).astype(f32)
    sg0 = lax.broadcasted_iota(jnp.int32, (_G, ng), 0)
    sg1 = lax.broadcasted_iota(jnp.int32, (_G, ng), 1)
    sumsel = (sg1 // _N == sg0).astype(f32)

    feats = jnp.dot(h_ref[...].reshape(ng, -1), win_ref[...],
                    preferred_element_type=f32)

    x4 = x_ref[...]
    xi = jnp.broadcast_to(x4[:, :, None, :], (_G, _N, _N, 3))
    xj = jnp.broadcast_to(x4[:, None, :, :], (_G, _N, _N, 3))
    diff = xj - xi
    d2 = jnp.sum(diff * diff, axis=3, keepdims=True)
    dist = jnp.sqrt(d2.reshape(_G * _E, 1) + 1e-12)
    rb = jnp.exp(-4.0 * (dist - centers) ** 2)

    rhid = jnp.maximum(
        jnp.dot(rb, w1cat_ref[...], preferred_element_type=f32)
        + b1cat_ref[...], 0.0)
    rkv = jnp.dot(rhid, wbd_ref[...], preferred_element_type=f32)
    rkv4 = rkv.reshape(_G, _N, _N, _L * 2 * _H)
    sts = [jnp.transpose(rkv4[g], (2, 0, 1)).reshape(_L * 2 * sh, _N)
           for g in range(_G)]

    for l in range(_L):
        q = jnp.dot(feats, wq_ref[l], preferred_element_type=f32)
        k = jnp.dot(feats, wk_ref[l], preferred_element_type=f32)
        v = jnp.dot(feats, wv_ref[l], preferred_element_type=f32)

        lst = jnp.concatenate(
            [jnp.dot(
                jnp.broadcast_to(q[g * _N:(g + 1) * _N][None],
                                 (_H, _N, _DH)).reshape(sh, _DH) * headmask,
                jnp.transpose(k[g * _N:(g + 1) * _N]),
                preferred_element_type=f32) for g in range(_G)],
            axis=0)
        base = l * 2 * sh
        rkst = jnp.concatenate(
            [sts[g][base:base + sh] for g in range(_G)], axis=0)
        rvst = jnp.concatenate(
            [sts[g][base + sh:base + 2 * sh] for g in range(_G)], axis=0)
        lst = lst * rkst + negdiag
        mx = jnp.max(lst, axis=1, keepdims=True)
        ex = jnp.exp(lst - mx)
        den = jnp.sum(ex, axis=1, keepdims=True)
        ast = ex / (den + 1e-9) * rvst

        agg = jnp.concatenate(
            [jnp.concatenate(
                [jnp.dot(ast[g * sh + hd * _N:g * sh + (hd + 1) * _N],
                         v[g * _N:(g + 1) * _N,
                           hd * _dvh:(hd + 1) * _dvh],
                         preferred_element_type=f32) for hd in range(_H)],
                axis=1) for g in range(_G)],
            axis=0)

        feats = feats + jnp.dot(agg, wo_ref[l],
                                preferred_element_type=f32) + bo_ref[l]
        mu = jnp.mean(feats, axis=-1, keepdims=True)
        xc = feats - mu
        var = jnp.mean(xc * xc, axis=-1, keepdims=True)
        feats = xc / jnp.sqrt(var + 1e-5) * g_ref[l] + b_ref[l]

    out = jnp.dot(feats, wfin_ref[...], preferred_element_type=f32)
    out = out + bfin_ref[...]
    res = jnp.dot(sumsel, out, preferred_element_type=f32)
    for g in range(_G):
        out_ref[g] = res[g:g + 1]


def kernel(h, x, n_atoms, params):
    Bsz = h.shape[0]
    f32 = jnp.float32
    layers = params["layers"]

    def stk(name):
        return jnp.stack([p[name] for p in layers])

    wq, wk, wv, wo = stk("Wq"), stk("Wk"), stk("Wv"), stk("Wo")
    bo = stk("bo").reshape(_L, 1, _DH)
    w1cat = jnp.concatenate([p["Wr1"] for p in layers], axis=1)
    b1cat = jnp.concatenate([p["br1"] for p in layers]).reshape(1, -1)
    rhid_n = layers[0]["Wr1"].shape[1]
    wbd = jnp.zeros((_L * rhid_n, _L * 2 * _H), f32)
    inv_sqrt_dh = 1.0 / (_dh ** 0.5)
    for l, p in enumerate(layers):
        blk = jnp.concatenate([p["Wrk"] * inv_sqrt_dh, p["Wrv"]], axis=1)
        wbd = wbd.at[l * rhid_n:(l + 1) * rhid_n,
                     l * 2 * _H:(l + 1) * 2 * _H].set(blk)
    gamma = stk("gamma").reshape(_L, 1, _DH)
    beta = stk("beta").reshape(_L, 1, _DH)
    bfin = params["b_fin"].reshape(1, _DH)

    def full(shape):
        return pl.BlockSpec(shape, lambda b: (0,) * len(shape))

    grid_spec = pl.GridSpec(
        grid=(Bsz // _G,),
        in_specs=[
            pl.BlockSpec((_G, _N, h.shape[2]), lambda b: (b, 0, 0)),
            pl.BlockSpec((_G, _N, 3), lambda b: (b, 0, 0)),
            full(params["W_in"].shape),
            full(wq.shape), full(wk.shape), full(wv.shape), full(wo.shape),
            full(bo.shape), full(w1cat.shape), full(b1cat.shape),
            full(wbd.shape), full(gamma.shape),
            full(beta.shape), full(params["W_fin"].shape), full(bfin.shape),
        ],
        out_specs=pl.BlockSpec((_G, 1, _DH), lambda b: (b, 0, 0)),
    )

    out = pl.pallas_call(
        _se3_body,
        grid_spec=grid_spec,
        out_shape=jax.ShapeDtypeStruct((Bsz, 1, _DH), f32),
    )(h, x, params["W_in"], wq, wk, wv, wo, bo, w1cat, b1cat, wbd,
      gamma, beta, params["W_fin"], bfin)

    return out.reshape(Bsz, _DH) / jnp.asarray(n_atoms, f32)


# speedup vs baseline: 595.3147x; 1.5999x over previous
import jax
import jax.numpy as jnp
from jax import lax
from jax.experimental import pallas as pl

_N = 48
_E = _N * _N
_DH = 128
_DV = 64
_H = 8
_dh = _DH // _H
_dvh = _DV // _H
_NRBF = 16
_L = 4
_G = 8


def _se3_body(h_ref, xt_ref, win_ref, wq_ref, wk_ref, wv_ref, wo_ref,
              bo_ref, w1catt_ref, b1col_ref, wbdt_ref, repsel_ref,
              tilesel_ref, g_ref, b_ref, wfin_ref, bfin_ref, out_ref):
    f32 = jnp.float32
    ng = _G * _N
    sh = _H * _N
    centers_col = lax.broadcasted_iota(
        jnp.int32, (_NRBF, 1), 0).astype(f32) * (5.0 / (_NRBF - 1))
    si = lax.broadcasted_iota(jnp.int32, (_G * sh, _N), 0)
    sj = lax.broadcasted_iota(jnp.int32, (_G * sh, _N), 1)
    negdiag = jnp.where((si // _H) % _N == sj, -1e30, 0.0)
    hm0 = lax.broadcasted_iota(jnp.int32, (sh, _DH), 0)
    hm1 = lax.broadcasted_iota(jnp.int32, (sh, _DH), 1)
    headmask = (hm0 % _H == hm1 // _dh).astype(f32)
    sm0 = lax.broadcasted_iota(jnp.int32, (sh, _DV), 0)
    sm1 = lax.broadcasted_iota(jnp.int32, (sh, _DV), 1)
    selmask = (sm0 % _H == sm1 // _dvh).astype(f32)
    sg0 = lax.broadcasted_iota(jnp.int32, (_G, ng), 0)
    sg1 = lax.broadcasted_iota(jnp.int32, (_G, ng), 1)
    sumsel = (sg1 // _N == sg0).astype(f32)

    feats = jnp.dot(h_ref[...].reshape(ng, -1), win_ref[...],
                    preferred_element_type=f32)

    sts = []
    for g in range(_G):
        xtg = xt_ref[g]
        xit = jnp.dot(xtg, repsel_ref[...],
                      preferred_element_type=f32)
        xjt = jnp.dot(xtg, tilesel_ref[...],
                      preferred_element_type=f32)
        dt = xjt - xit
        d2t = jnp.sum(dt * dt, axis=0, keepdims=True)
        distt = jnp.sqrt(d2t + 1e-12)
        rbt = jnp.exp(-4.0 * (distt - centers_col) ** 2)
        rhidt = jnp.maximum(
            jnp.dot(w1catt_ref[...], rbt, preferred_element_type=f32)
            + b1col_ref[...], 0.0)
        rkvt = jnp.dot(wbdt_ref[...], rhidt,
                       preferred_element_type=f32)
        rkv3 = jnp.transpose(rkvt).reshape(_N, _N, _L * 2 * _H)
        sts.append(jnp.transpose(rkv3, (0, 2, 1)))

    for l in range(_L):
        q = jnp.dot(feats, wq_ref[l], preferred_element_type=f32)
        k = jnp.dot(feats, wk_ref[l], preferred_element_type=f32)
        v = jnp.dot(feats, wv_ref[l], preferred_element_type=f32)

        lst = jnp.concatenate(
            [jnp.dot(
                jnp.broadcast_to(q[g * _N:(g + 1) * _N][:, None],
                                 (_N, _H, _DH)).reshape(sh, _DH) * headmask,
                jnp.transpose(k[g * _N:(g + 1) * _N]),
                preferred_element_type=f32) for g in range(_G)],
            axis=0)
        c0 = l * 2 * _H
        rkst = jnp.concatenate(
            [sts[g][:, c0:c0 + _H, :].reshape(sh, _N)
             for g in range(_G)], axis=0)
        rvst = jnp.concatenate(
            [sts[g][:, c0 + _H:c0 + 2 * _H, :].reshape(sh, _N)
             for g in range(_G)], axis=0)
        lst = lst * rkst + negdiag
        mx = jnp.max(lst, axis=1, keepdims=True)
        ex = jnp.exp(lst - mx)
        den = jnp.sum(ex, axis=1, keepdims=True)
        ast = ex / (den + 1e-9) * rvst

        agg = jnp.concatenate(
            [jnp.sum(
                (jnp.dot(ast[g * sh:(g + 1) * sh],
                         v[g * _N:(g + 1) * _N],
                         preferred_element_type=f32) * selmask
                 ).reshape(_N, _H, _DV),
                axis=1) for g in range(_G)],
            axis=0)

        feats = feats + jnp.dot(agg, wo_ref[l],
                                preferred_element_type=f32) + bo_ref[l]
        mu = jnp.mean(feats, axis=-1, keepdims=True)
        xc = feats - mu
        var = jnp.mean(xc * xc, axis=-1, keepdims=True)
        feats = xc / jnp.sqrt(var + 1e-5) * g_ref[l] + b_ref[l]

    out = jnp.dot(feats, wfin_ref[...], preferred_element_type=f32)
    out = out + bfin_ref[...]
    res = jnp.dot(sumsel, out, preferred_element_type=f32)
    for g in range(_G):
        out_ref[g] = res[g:g + 1]


def kernel(h, x, n_atoms, params):
    Bsz = h.shape[0]
    f32 = jnp.float32
    layers = params["layers"]

    def stk(name):
        return jnp.stack([p[name] for p in layers])

    wq, wk, wv, wo = stk("Wq"), stk("Wk"), stk("Wv"), stk("Wo")
    bo = stk("bo").reshape(_L, 1, _DH)
    w1cat = jnp.concatenate([p["Wr1"] for p in layers], axis=1)
    b1cat = jnp.concatenate([p["br1"] for p in layers]).reshape(1, -1)
    rhid_n = layers[0]["Wr1"].shape[1]
    wbd = jnp.zeros((_L * rhid_n, _L * 2 * _H), f32)
    inv_sqrt_dh = 1.0 / (_dh ** 0.5)
    for l, p in enumerate(layers):
        blk = jnp.concatenate([p["Wrk"] * inv_sqrt_dh, p["Wrv"]], axis=1)
        wbd = wbd.at[l * rhid_n:(l + 1) * rhid_n,
                     l * 2 * _H:(l + 1) * 2 * _H].set(blk)
    gamma = stk("gamma").reshape(_L, 1, _DH)
    beta = stk("beta").reshape(_L, 1, _DH)
    bfin = params["b_fin"].reshape(1, _DH)
    w1catt = w1cat.T
    b1col = b1cat.reshape(-1, 1)
    wbdt = wbd.T
    ev = jnp.arange(_E)
    av = jnp.arange(_N)[:, None]
    repsel = (ev[None, :] // _N == av).astype(f32)
    tilesel = (ev[None, :] % _N == av).astype(f32)

    def full(shape):
        return pl.BlockSpec(shape, lambda b: (0,) * len(shape))

    grid_spec = pl.GridSpec(
        grid=(Bsz // _G,),
        in_specs=[
            pl.BlockSpec((_G, _N, h.shape[2]), lambda b: (b, 0, 0)),
            pl.BlockSpec((_G, 3, _N), lambda b: (b, 0, 0)),
            full(params["W_in"].shape),
            full(wq.shape), full(wk.shape), full(wv.shape), full(wo.shape),
            full(bo.shape), full(w1catt.shape), full(b1col.shape),
            full(wbdt.shape), full(repsel.shape), full(tilesel.shape),
            full(gamma.shape),
            full(beta.shape), full(params["W_fin"].shape), full(bfin.shape),
        ],
        out_specs=pl.BlockSpec((_G, 1, _DH), lambda b: (b, 0, 0)),
    )

    out = pl.pallas_call(
        _se3_body,
        grid_spec=grid_spec,
        out_shape=jax.ShapeDtypeStruct((Bsz, 1, _DH), f32),
    )(h, jnp.swapaxes(x, 1, 2), params["W_in"], wq, wk, wv, wo, bo,
      w1catt, b1col, wbdt, repsel, tilesel,
      gamma, beta, params["W_fin"], bfin)

    return out.reshape(Bsz, _DH) / jnp.asarray(n_atoms, f32)
